# baseline (speedup 1.0000x reference)
import functools

import jax
import jax.numpy as jnp
from jax import lax
from jax.experimental import pallas as pl
from jax.experimental.pallas import tpu as pltpu
from jax.experimental.pallas import tpu_sc as plsc

N = 10000
E = 320000
D = 128
C = 64

NW = 32
NPT = 320
NPAD = NW * NPT
P = 16384
SUB = 2048
G = 128


def _segmax_body(m_hbm, src_hbm, dst_hbm, agg_hbm,
                 src_in, dst_in, comp_src, comp_ldst, rows, rows1, agg, gsem, gsem1):
    cid = lax.axis_index("c")
    sid = lax.axis_index("s")
    wid = sid * 2 + cid
    base = wid * NPT
    lanes = lax.iota(jnp.int32, 16)

    def zrow(i, c):
        for fv in range(D // 16):
            agg[i, pl.ds(fv * 16, 16)] = jnp.zeros((16,), jnp.float32)
        return c
    lax.fori_loop(0, NPT + 1, zrow, 0)

    def pass_body(p, c):
        def sub_body(s, w):
            off = p * P + s * SUB
            pltpu.sync_copy(src_hbm.at[pl.ds(off, SUB)], src_in)
            pltpu.sync_copy(dst_hbm.at[pl.ds(off, SUB)], dst_in)

            def vbody(v, w):
                sv = src_in[pl.ds(v * 16, 16)]
                dv = dst_in[pl.ds(v * 16, 16)]
                ld = dv - base
                mask = (ld >= 0) & (ld < NPT)
                mi = mask.astype(jnp.int32)
                pos = w + lanes
                pos = jnp.where(mask, pos, 0)
                plsc.store_scatter(comp_src, [pos], sv, mask=mask)
                plsc.store_scatter(comp_ldst, [pos], ld, mask=mask)
                return w + plsc.all_reduce_population_count(mask)[0]

            return lax.fori_loop(0, SUB // 16, vbody, w)

        w = lax.fori_loop(0, P // SUB, sub_body, jnp.int32(0))

        zero16 = jnp.zeros((16,), jnp.int32)
        dummy16 = jnp.full((16,), NPT, jnp.int32)
        for k in range(2 * G // 16):
            idxs = w + k * 16 + lanes
            plsc.store_scatter(comp_src, [idxs], zero16)
            plsc.store_scatter(comp_ldst, [idxs], dummy16)
        npairs = (w + (2 * G - 1)) // (2 * G)

        def fire_g(g, rb, sem):
            pltpu.async_copy(m_hbm.at[comp_src.at[pl.ds(g * G, G)]], rb, sem)

        def drain_g(g, rb, sem):
            pltpu.make_async_copy(m_hbm.at[comp_src.at[pl.ds(g * G, G)]],
                                  rb, sem).wait()

        def process_g(g, rb, c):
            goff = g * G

            def ebody(e, c):
                le = comp_ldst[pl.ds(goff + e, 16)][0]
                for fv in range(D // 16):
                    sl = pl.ds(fv * 16, 16)
                    agg[le, sl] = jnp.maximum(agg[le, sl], rb[e, sl])
                return c

            return lax.fori_loop(0, G, ebody, c)

        def pair_body(i, c):
            pltpu.async_copy(m_hbm.at[comp_src.at[pl.ds(2 * i * G, G)]],
                             rows, gsem).wait()
            c = process_g(2 * i, rows, c)
            pltpu.async_copy(m_hbm.at[comp_src.at[pl.ds((2 * i + 1) * G, G)]],
                             rows1, gsem1).wait()
            return process_g(2 * i + 1, rows1, c)

        return lax.fori_loop(0, npairs, pair_body, c)

    lax.fori_loop(0, E // P, pass_body, 0)

    pltpu.sync_copy(agg.at[pl.ds(0, NPT)], agg_hbm.at[pl.ds(base, NPT)])


_segmax = pl.kernel(
    _segmax_body,
    out_type=jax.ShapeDtypeStruct((NPAD, D), jnp.float32),
    mesh=plsc.VectorSubcoreMesh(core_axis_name="c", subcore_axis_name="s"),
    scratch_types=[
        pltpu.VMEM((SUB,), jnp.int32),
        pltpu.VMEM((SUB,), jnp.int32),
        pltpu.VMEM((P + 2 * G + 16,), jnp.int32),
        pltpu.VMEM((P + 2 * G + 16,), jnp.int32),
        pltpu.VMEM((G, D), jnp.float32),
        pltpu.VMEM((G, D), jnp.float32),
        pltpu.VMEM((NPT + 1, D), jnp.float32),
        pltpu.SemaphoreType.DMA,
        pltpu.SemaphoreType.DMA,
    ],
)


_BLK = 1000
_GRID = N // _BLK


def _dotT(a, w):
    return lax.dot_general(a, w, (((1,), (1,)), ((), ())),
                           preferred_element_type=jnp.float32)


def _dense1_body(x_ref, wp_ref, bp_ref, ws_ref, m_ref, s_ref):
    xb = x_ref[...]
    m_ref[...] = jnp.maximum(_dotT(xb, wp_ref[...]) + bp_ref[...], 0.0)
    s_ref[...] = _dotT(xb, ws_ref[...])


_dense1 = pl.pallas_call(
    _dense1_body,
    grid=(_GRID,),
    in_specs=[
        pl.BlockSpec((_BLK, D), lambda i: (i, 0)),
        pl.BlockSpec((D, D), lambda i: (0, 0)),
        pl.BlockSpec((1, D), lambda i: (0, 0)),
        pl.BlockSpec((D, D), lambda i: (0, 0)),
    ],
    out_specs=(
        pl.BlockSpec((_BLK, D), lambda i: (i, 0)),
        pl.BlockSpec((_BLK, D), lambda i: (i, 0)),
    ),
    out_shape=(
        jax.ShapeDtypeStruct((N, D), jnp.float32),
        jax.ShapeDtypeStruct((N, D), jnp.float32),
    ),
)


def _dense2_body(s1_ref, agg_ref, wn1_ref, bn1_ref, wp2_ref, bp2_ref,
                 ws2_ref, m2_ref, s2_ref):
    h = s1_ref[...] + _dotT(agg_ref[...], wn1_ref[...]) + bn1_ref[...]
    h = jnp.maximum(h, 0.0)
    m2_ref[...] = jnp.maximum(_dotT(h, wp2_ref[...]) + bp2_ref[...], 0.0)
    s2_ref[...] = _dotT(h, ws2_ref[...])


_dense2 = pl.pallas_call(
    _dense2_body,
    grid=(_GRID,),
    in_specs=[
        pl.BlockSpec((_BLK, D), lambda i: (i, 0)),
        pl.BlockSpec((_BLK, D), lambda i: (i, 0)),
        pl.BlockSpec((D, D), lambda i: (0, 0)),
        pl.BlockSpec((1, D), lambda i: (0, 0)),
        pl.BlockSpec((D, D), lambda i: (0, 0)),
        pl.BlockSpec((1, D), lambda i: (0, 0)),
        pl.BlockSpec((C, D), lambda i: (0, 0)),
    ],
    out_specs=(
        pl.BlockSpec((_BLK, D), lambda i: (i, 0)),
        pl.BlockSpec((_BLK, C), lambda i: (i, 0)),
    ),
    out_shape=(
        jax.ShapeDtypeStruct((N, D), jnp.float32),
        jax.ShapeDtypeStruct((N, C), jnp.float32),
    ),
)


def _dense3_body(s2_ref, agg_ref, wn2_ref, bn2_ref, o_ref):
    o_ref[...] = s2_ref[...] + _dotT(agg_ref[...], wn2_ref[...]) + bn2_ref[...]


_dense3 = pl.pallas_call(
    _dense3_body,
    grid=(_GRID,),
    in_specs=[
        pl.BlockSpec((_BLK, C), lambda i: (i, 0)),
        pl.BlockSpec((_BLK, D), lambda i: (i, 0)),
        pl.BlockSpec((C, D), lambda i: (0, 0)),
        pl.BlockSpec((1, C), lambda i: (0, 0)),
    ],
    out_specs=pl.BlockSpec((_BLK, C), lambda i: (i, 0)),
    out_shape=jax.ShapeDtypeStruct((N, C), jnp.float32),
)


@jax.jit
def kernel(in_feat, edge_index, W_pool1, b_pool1, W_self1, W_neigh1, b_neigh1,
           W_pool2, b_pool2, W_self2, W_neigh2, b_neigh2):
    src = edge_index[0].astype(jnp.int32)
    dst = edge_index[1].astype(jnp.int32)

    m1, s1 = _dense1(in_feat, W_pool1, b_pool1.reshape(1, D), W_self1)
    agg1 = _segmax(m1, src, dst)[:N]
    m2, s2 = _dense2(s1, agg1, W_neigh1, b_neigh1.reshape(1, D),
                     W_pool2, b_pool2.reshape(1, D), W_self2)
    agg2 = _segmax(m2, src, dst)[:N]
    return _dense3(s2, agg2, W_neigh2, b_neigh2.reshape(1, C))


# speedup vs baseline: 1.7995x; 1.7995x over previous
import functools

import jax
import jax.numpy as jnp
from jax import lax
from jax.experimental import pallas as pl
from jax.experimental.pallas import tpu as pltpu
from jax.experimental.pallas import tpu_sc as plsc

N = 10000
E = 320000
D = 128
C = 64

NW = 32
NPT = 320
NPAD = NW * NPT
P = 16384
SUB = 2048
G = 128


def _segmax_body(m_hbm, src_hbm, dst_hbm, agg_hbm,
                 src_in, dst_in, comp_src, comp_ldst, rows, rows1, agg, gsem, gsem1):
    cid = lax.axis_index("c")
    sid = lax.axis_index("s")
    wid = sid * 2 + cid
    base = wid * NPT
    lanes = lax.iota(jnp.int32, 16)

    def zrow(i, c):
        for fv in range(D // 16):
            agg[i, pl.ds(fv * 16, 16)] = jnp.zeros((16,), jnp.float32)
        return c
    lax.fori_loop(0, NPT + 1, zrow, 0)

    def pass_body(p, c):
        def sub_body(s, w):
            off = p * P + s * SUB
            pltpu.sync_copy(src_hbm.at[pl.ds(off, SUB)], src_in)
            pltpu.sync_copy(dst_hbm.at[pl.ds(off, SUB)], dst_in)

            def vbody(v, w):
                sv = src_in[pl.ds(v * 16, 16)]
                dv = dst_in[pl.ds(v * 16, 16)]
                ld = dv - base
                mask = (ld >= 0) & (ld < NPT)
                mi = mask.astype(jnp.int32)
                pos = w + lanes
                pos = jnp.where(mask, pos, 0)
                plsc.store_scatter(comp_src, [pos], sv, mask=mask)
                plsc.store_scatter(comp_ldst, [pos], ld, mask=mask)
                return w + plsc.all_reduce_population_count(mask)[0]

            return lax.fori_loop(0, SUB // 16, vbody, w)

        w = lax.fori_loop(0, P // SUB, sub_body, jnp.int32(0))

        zero16 = jnp.zeros((16,), jnp.int32)
        dummy16 = jnp.full((16,), NPT, jnp.int32)
        for k in range(G // 16):
            idxs = w + k * 16 + lanes
            plsc.store_scatter(comp_src, [idxs], zero16)
            plsc.store_scatter(comp_ldst, [idxs], dummy16)
        ngroups = (w + (G - 1)) // G

        def gbody(g, c):
            goff = g * G
            pltpu.async_copy(m_hbm.at[comp_src.at[pl.ds(goff, G)]],
                             rows, gsem).wait()

            def ebody(e, c):
                le = comp_ldst[pl.ds(goff + e, 16)][0]
                for fv in range(D // 16):
                    sl = pl.ds(fv * 16, 16)
                    agg[le, sl] = jnp.maximum(agg[le, sl], rows[e, sl])
                return c

            return lax.fori_loop(0, G, ebody, c)

        return lax.fori_loop(0, ngroups, gbody, c)

    lax.fori_loop(0, E // P, pass_body, 0)

    pltpu.sync_copy(agg.at[pl.ds(0, NPT)], agg_hbm.at[pl.ds(base, NPT)])


_segmax = pl.kernel(
    _segmax_body,
    out_type=jax.ShapeDtypeStruct((NPAD, D), jnp.float32),
    mesh=plsc.VectorSubcoreMesh(core_axis_name="c", subcore_axis_name="s"),
    scratch_types=[
        pltpu.VMEM((SUB,), jnp.int32),
        pltpu.VMEM((SUB,), jnp.int32),
        pltpu.VMEM((P + 2 * G + 16,), jnp.int32),
        pltpu.VMEM((P + 2 * G + 16,), jnp.int32),
        pltpu.VMEM((G, D), jnp.float32),
        pltpu.VMEM((G, D), jnp.float32),
        pltpu.VMEM((NPT + 1, D), jnp.float32),
        pltpu.SemaphoreType.DMA,
        pltpu.SemaphoreType.DMA,
    ],
)


_BLK = 1000
_GRID = N // _BLK


def _dotT(a, w):
    return lax.dot_general(a, w, (((1,), (1,)), ((), ())),
                           preferred_element_type=jnp.float32)


def _dense1_body(x_ref, wp_ref, bp_ref, ws_ref, m_ref, s_ref):
    xb = x_ref[...]
    m_ref[...] = jnp.maximum(_dotT(xb, wp_ref[...]) + bp_ref[...], 0.0)
    s_ref[...] = _dotT(xb, ws_ref[...])


_dense1 = pl.pallas_call(
    _dense1_body,
    grid=(_GRID,),
    in_specs=[
        pl.BlockSpec((_BLK, D), lambda i: (i, 0)),
        pl.BlockSpec((D, D), lambda i: (0, 0)),
        pl.BlockSpec((1, D), lambda i: (0, 0)),
        pl.BlockSpec((D, D), lambda i: (0, 0)),
    ],
    out_specs=(
        pl.BlockSpec((_BLK, D), lambda i: (i, 0)),
        pl.BlockSpec((_BLK, D), lambda i: (i, 0)),
    ),
    out_shape=(
        jax.ShapeDtypeStruct((N, D), jnp.float32),
        jax.ShapeDtypeStruct((N, D), jnp.float32),
    ),
)


def _dense2_body(s1_ref, agg_ref, wn1_ref, bn1_ref, wp2_ref, bp2_ref,
                 ws2_ref, m2_ref, s2_ref):
    h = s1_ref[...] + _dotT(agg_ref[...], wn1_ref[...]) + bn1_ref[...]
    h = jnp.maximum(h, 0.0)
    m2_ref[...] = jnp.maximum(_dotT(h, wp2_ref[...]) + bp2_ref[...], 0.0)
    s2_ref[...] = _dotT(h, ws2_ref[...])


_dense2 = pl.pallas_call(
    _dense2_body,
    grid=(_GRID,),
    in_specs=[
        pl.BlockSpec((_BLK, D), lambda i: (i, 0)),
        pl.BlockSpec((_BLK, D), lambda i: (i, 0)),
        pl.BlockSpec((D, D), lambda i: (0, 0)),
        pl.BlockSpec((1, D), lambda i: (0, 0)),
        pl.BlockSpec((D, D), lambda i: (0, 0)),
        pl.BlockSpec((1, D), lambda i: (0, 0)),
        pl.BlockSpec((C, D), lambda i: (0, 0)),
    ],
    out_specs=(
        pl.BlockSpec((_BLK, D), lambda i: (i, 0)),
        pl.BlockSpec((_BLK, C), lambda i: (i, 0)),
    ),
    out_shape=(
        jax.ShapeDtypeStruct((N, D), jnp.float32),
        jax.ShapeDtypeStruct((N, C), jnp.float32),
    ),
)


def _dense3_body(s2_ref, agg_ref, wn2_ref, bn2_ref, o_ref):
    o_ref[...] = s2_ref[...] + _dotT(agg_ref[...], wn2_ref[...]) + bn2_ref[...]


_dense3 = pl.pallas_call(
    _dense3_body,
    grid=(_GRID,),
    in_specs=[
        pl.BlockSpec((_BLK, C), lambda i: (i, 0)),
        pl.BlockSpec((_BLK, D), lambda i: (i, 0)),
        pl.BlockSpec((C, D), lambda i: (0, 0)),
        pl.BlockSpec((1, C), lambda i: (0, 0)),
    ],
    out_specs=pl.BlockSpec((_BLK, C), lambda i: (i, 0)),
    out_shape=jax.ShapeDtypeStruct((N, C), jnp.float32),
)


@jax.jit
def kernel(in_feat, edge_index, W_pool1, b_pool1, W_self1, W_neigh1, b_neigh1,
           W_pool2, b_pool2, W_self2, W_neigh2, b_neigh2):
    src = edge_index[0].astype(jnp.int32)
    dst = edge_index[1].astype(jnp.int32)

    m1, s1 = _dense1(in_feat, W_pool1, b_pool1.reshape(1, D), W_self1)
    agg1 = _segmax(m1, src, dst)[:N]
    m2, s2 = _dense2(s1, agg1, W_neigh1, b_neigh1.reshape(1, D),
                     W_pool2, b_pool2.reshape(1, D), W_self2)
    agg2 = _segmax(m2, src, dst)[:N]
    return _dense3(s2, agg2, W_neigh2, b_neigh2.reshape(1, C))


# speedup vs baseline: 1.8021x; 1.0014x over previous
import functools

import jax
import jax.numpy as jnp
from jax import lax
from jax.experimental import pallas as pl
from jax.experimental.pallas import tpu as pltpu
from jax.experimental.pallas import tpu_sc as plsc

N = 10000
E = 320000
D = 128
C = 64

NW = 32
NPT = 320
NPAD = NW * NPT
P = 16384
SUB = 2048
G = 128


def _segmax_body(m_hbm, src_hbm, dst_hbm, agg_hbm,
                 src_in, dst_in, comp_src, comp_ldst, rows, agg, gsem):
    cid = lax.axis_index("c")
    sid = lax.axis_index("s")
    wid = sid * 2 + cid
    base = wid * NPT
    lanes = lax.iota(jnp.int32, 16)

    def zrow(i, c):
        for fv in range(D // 16):
            agg[i, pl.ds(fv * 16, 16)] = jnp.zeros((16,), jnp.float32)
        return c
    lax.fori_loop(0, NPT + 1, zrow, 0)

    def pass_body(p, c):
        def sub_body(s, w):
            off = p * P + s * SUB
            pltpu.sync_copy(src_hbm.at[pl.ds(off, SUB)], src_in)
            pltpu.sync_copy(dst_hbm.at[pl.ds(off, SUB)], dst_in)

            def vbody(v, w):
                sv = src_in[pl.ds(v * 16, 16)]
                dv = dst_in[pl.ds(v * 16, 16)]
                ld = dv - base
                mask = (ld >= 0) & (ld < NPT)
                mi = mask.astype(jnp.int32)
                pos = w + lanes
                pos = jnp.where(mask, pos, 0)
                plsc.store_scatter(comp_src, [pos], sv, mask=mask)
                plsc.store_scatter(comp_ldst, [pos], ld, mask=mask)
                return w + plsc.all_reduce_population_count(mask)[0]

            return lax.fori_loop(0, SUB // 16, vbody, w)

        w = lax.fori_loop(0, P // SUB, sub_body, jnp.int32(0))

        zero16 = jnp.zeros((16,), jnp.int32)
        dummy16 = jnp.full((16,), NPT, jnp.int32)
        for k in range(G // 16):
            idxs = w + k * 16 + lanes
            plsc.store_scatter(comp_src, [idxs], zero16)
            plsc.store_scatter(comp_ldst, [idxs], dummy16)
        ngroups = (w + (G - 1)) // G

        def gbody(g, c):
            goff = g * G
            pltpu.async_copy(m_hbm.at[comp_src.at[pl.ds(goff, G)]],
                             rows, gsem).wait()

            def ebody(e, c):
                le = comp_ldst[pl.ds(goff + e, 16)][0]
                for fv in range(D // 16):
                    sl = pl.ds(fv * 16, 16)
                    agg[le, sl] = jnp.maximum(agg[le, sl], rows[e, sl])
                return c

            return lax.fori_loop(0, G, ebody, c)

        return lax.fori_loop(0, ngroups, gbody, c)

    lax.fori_loop(0, E // P, pass_body, 0)

    pltpu.sync_copy(agg.at[pl.ds(0, NPT)], agg_hbm.at[pl.ds(base, NPT)])


_segmax = pl.kernel(
    _segmax_body,
    out_type=jax.ShapeDtypeStruct((NPAD, D), jnp.float32),
    mesh=plsc.VectorSubcoreMesh(core_axis_name="c", subcore_axis_name="s"),
    scratch_types=[
        pltpu.VMEM((SUB,), jnp.int32),
        pltpu.VMEM((SUB,), jnp.int32),
        pltpu.VMEM((P + G + 16,), jnp.int32),
        pltpu.VMEM((P + G + 16,), jnp.int32),
        pltpu.VMEM((G, D), jnp.float32),
        pltpu.VMEM((NPT + 1, D), jnp.float32),
        pltpu.SemaphoreType.DMA,
    ],
)


_BLK = 1000
_GRID = N // _BLK


def _dotT(a, w):
    return lax.dot_general(a, w, (((1,), (1,)), ((), ())),
                           preferred_element_type=jnp.float32)


def _dense1_body(x_ref, wp_ref, bp_ref, ws_ref, m_ref, s_ref):
    xb = x_ref[...]
    m_ref[...] = jnp.maximum(_dotT(xb, wp_ref[...]) + bp_ref[...], 0.0)
    s_ref[...] = _dotT(xb, ws_ref[...])


_dense1 = pl.pallas_call(
    _dense1_body,
    grid=(_GRID,),
    in_specs=[
        pl.BlockSpec((_BLK, D), lambda i: (i, 0)),
        pl.BlockSpec((D, D), lambda i: (0, 0)),
        pl.BlockSpec((1, D), lambda i: (0, 0)),
        pl.BlockSpec((D, D), lambda i: (0, 0)),
    ],
    out_specs=(
        pl.BlockSpec((_BLK, D), lambda i: (i, 0)),
        pl.BlockSpec((_BLK, D), lambda i: (i, 0)),
    ),
    out_shape=(
        jax.ShapeDtypeStruct((N, D), jnp.float32),
        jax.ShapeDtypeStruct((N, D), jnp.float32),
    ),
)


def _dense2_body(s1_ref, agg_ref, wn1_ref, bn1_ref, wp2_ref, bp2_ref,
                 ws2_ref, m2_ref, s2_ref):
    h = s1_ref[...] + _dotT(agg_ref[...], wn1_ref[...]) + bn1_ref[...]
    h = jnp.maximum(h, 0.0)
    m2_ref[...] = jnp.maximum(_dotT(h, wp2_ref[...]) + bp2_ref[...], 0.0)
    s2_ref[...] = _dotT(h, ws2_ref[...])


_dense2 = pl.pallas_call(
    _dense2_body,
    grid=(_GRID,),
    in_specs=[
        pl.BlockSpec((_BLK, D), lambda i: (i, 0)),
        pl.BlockSpec((_BLK, D), lambda i: (i, 0)),
        pl.BlockSpec((D, D), lambda i: (0, 0)),
        pl.BlockSpec((1, D), lambda i: (0, 0)),
        pl.BlockSpec((D, D), lambda i: (0, 0)),
        pl.BlockSpec((1, D), lambda i: (0, 0)),
        pl.BlockSpec((C, D), lambda i: (0, 0)),
    ],
    out_specs=(
        pl.BlockSpec((_BLK, D), lambda i: (i, 0)),
        pl.BlockSpec((_BLK, C), lambda i: (i, 0)),
    ),
    out_shape=(
        jax.ShapeDtypeStruct((N, D), jnp.float32),
        jax.ShapeDtypeStruct((N, C), jnp.float32),
    ),
)


def _dense3_body(s2_ref, agg_ref, wn2_ref, bn2_ref, o_ref):
    o_ref[...] = s2_ref[...] + _dotT(agg_ref[...], wn2_ref[...]) + bn2_ref[...]


_dense3 = pl.pallas_call(
    _dense3_body,
    grid=(_GRID,),
    in_specs=[
        pl.BlockSpec((_BLK, C), lambda i: (i, 0)),
        pl.BlockSpec((_BLK, D), lambda i: (i, 0)),
        pl.BlockSpec((C, D), lambda i: (0, 0)),
        pl.BlockSpec((1, C), lambda i: (0, 0)),
    ],
    out_specs=pl.BlockSpec((_BLK, C), lambda i: (i, 0)),
    out_shape=jax.ShapeDtypeStruct((N, C), jnp.float32),
)


@jax.jit
def kernel(in_feat, edge_index, W_pool1, b_pool1, W_self1, W_neigh1, b_neigh1,
           W_pool2, b_pool2, W_self2, W_neigh2, b_neigh2):
    src = edge_index[0].astype(jnp.int32)
    dst = edge_index[1].astype(jnp.int32)

    m1, s1 = _dense1(in_feat, W_pool1, b_pool1.reshape(1, D), W_self1)
    agg1 = _segmax(m1, src, dst)[:N]
    m2, s2 = _dense2(s1, agg1, W_neigh1, b_neigh1.reshape(1, D),
                     W_pool2, b_pool2.reshape(1, D), W_self2)
    agg2 = _segmax(m2, src, dst)[:N]
    return _dense3(s2, agg2, W_neigh2, b_neigh2.reshape(1, C))


# speedup vs baseline: 2.5266x; 1.4020x over previous
import jax
import jax.numpy as jnp
from jax import lax
from jax.experimental import pallas as pl
from jax.experimental.pallas import tpu as pltpu
from jax.experimental.pallas import tpu_sc as plsc

N = 10000
NP = 10240
E = 320000
D = 128
C = 64

NW = 32
F = D // NW
FB = F * NP
SUB = 2000


def _segmax_body(m_hbm, src_hbm, dst_hbm, agg_hbm,
                 src_in, dst_in, m_l, agg_l, tmp):
    cid = lax.axis_index("c")
    sid = lax.axis_index("s")
    wid = sid * 2 + cid
    lanes = lax.iota(jnp.int32, 16)

    moff = pl.multiple_of(wid * FB, FB)
    pltpu.sync_copy(m_hbm.at[pl.ds(moff, FB)], m_l)

    def zbody(i, c):
        agg_l[pl.ds(i * 16, 16)] = jnp.zeros((16,), jnp.float32)
        return c
    lax.fori_loop(0, FB // 16, zbody, 0)

    def chunk_body(t, c):
        off = pl.multiple_of(t * SUB, SUB)
        pltpu.sync_copy(src_hbm.at[pl.ds(off, SUB)], src_in)
        pltpu.sync_copy(dst_hbm.at[pl.ds(off, SUB)], dst_in)

        def vbody(v, c):
            sv = src_in[pl.ds(v * 16, 16)]
            dv = dst_in[pl.ds(v * 16, 16)]

            plsc.store_scatter(tmp, [dv], lanes)
            rd = plsc.load_gather(tmp, [dv])
            nodup = plsc.all_reduce_population_count(rd != lanes)[0] == 0

            vals = [plsc.load_gather(m_l, [f * NP + sv]) for f in range(F)]
            idxd = [f * NP + dv for f in range(F)]

            @pl.when(nodup)
            def _():
                for f in range(F):
                    cur = plsc.load_gather(agg_l, [idxd[f]])
                    plsc.store_scatter(agg_l, [idxd[f]],
                                       jnp.maximum(cur, vals[f]))

            @pl.when(jnp.logical_not(nodup))
            def _():
                for j in range(16):
                    msk = lanes == j
                    for f in range(F):
                        cur = plsc.load_gather(agg_l, [idxd[f]], mask=msk)
                        nv = jnp.maximum(cur, vals[f])
                        plsc.store_scatter(agg_l, [idxd[f]], nv, mask=msk)

            return c

        return lax.fori_loop(0, SUB // 16, vbody, c)

    lax.fori_loop(0, E // SUB, chunk_body, 0)

    pltpu.sync_copy(agg_l, agg_hbm.at[pl.ds(moff, FB)])


_segmax = pl.kernel(
    _segmax_body,
    out_type=jax.ShapeDtypeStruct((NW * FB,), jnp.float32),
    mesh=plsc.VectorSubcoreMesh(core_axis_name="c", subcore_axis_name="s"),
    compiler_params=pltpu.CompilerParams(needs_layout_passes=False),
    scratch_types=[
        pltpu.VMEM((SUB,), jnp.int32),
        pltpu.VMEM((SUB,), jnp.int32),
        pltpu.VMEM((FB,), jnp.float32),
        pltpu.VMEM((FB,), jnp.float32),
        pltpu.VMEM((NP,), jnp.int32),
    ],
)


_BLK = 1024
_GRID = NP // _BLK


def _dot(w, x):
    return lax.dot_general(w, x, (((1,), (0,)), ((), ())),
                           preferred_element_type=jnp.float32)


def _dotT(w, x):
    return lax.dot_general(w, x, (((1,), (1,)), ((), ())),
                           preferred_element_type=jnp.float32)


def _dense1_body(x_ref, wp_ref, bp_ref, ws_ref, m_ref, s_ref):
    xb = x_ref[...]
    m_ref[...] = jnp.maximum(_dotT(wp_ref[...], xb) + bp_ref[...], 0.0)
    s_ref[...] = _dotT(ws_ref[...], xb)


_dense1 = pl.pallas_call(
    _dense1_body,
    grid=(_GRID,),
    in_specs=[
        pl.BlockSpec((_BLK, D), lambda i: (i, 0)),
        pl.BlockSpec((D, D), lambda i: (0, 0)),
        pl.BlockSpec((D, 1), lambda i: (0, 0)),
        pl.BlockSpec((D, D), lambda i: (0, 0)),
    ],
    out_specs=(
        pl.BlockSpec((D, _BLK), lambda i: (0, i)),
        pl.BlockSpec((D, _BLK), lambda i: (0, i)),
    ),
    out_shape=(
        jax.ShapeDtypeStruct((D, NP), jnp.float32),
        jax.ShapeDtypeStruct((D, NP), jnp.float32),
    ),
)


def _dense2_body(s1_ref, agg_ref, wn1_ref, bn1_ref, wp2_ref, bp2_ref,
                 ws2_ref, m2_ref, s2_ref):
    h = s1_ref[...] + _dot(wn1_ref[...], agg_ref[...]) + bn1_ref[...]
    h = jnp.maximum(h, 0.0)
    m2_ref[...] = jnp.maximum(_dot(wp2_ref[...], h) + bp2_ref[...], 0.0)
    s2_ref[...] = _dot(ws2_ref[...], h)


_dense2 = pl.pallas_call(
    _dense2_body,
    grid=(_GRID,),
    in_specs=[
        pl.BlockSpec((D, _BLK), lambda i: (0, i)),
        pl.BlockSpec((D, _BLK), lambda i: (0, i)),
        pl.BlockSpec((D, D), lambda i: (0, 0)),
        pl.BlockSpec((D, 1), lambda i: (0, 0)),
        pl.BlockSpec((D, D), lambda i: (0, 0)),
        pl.BlockSpec((D, 1), lambda i: (0, 0)),
        pl.BlockSpec((C, D), lambda i: (0, 0)),
    ],
    out_specs=(
        pl.BlockSpec((D, _BLK), lambda i: (0, i)),
        pl.BlockSpec((C, _BLK), lambda i: (0, i)),
    ),
    out_shape=(
        jax.ShapeDtypeStruct((D, NP), jnp.float32),
        jax.ShapeDtypeStruct((C, NP), jnp.float32),
    ),
)


def _dense3_body(s2_ref, agg_ref, wn2_ref, bn2_ref, o_ref):
    ot = s2_ref[...] + _dot(wn2_ref[...], agg_ref[...]) + bn2_ref[...]
    o_ref[...] = ot.T


_dense3 = pl.pallas_call(
    _dense3_body,
    grid=(_GRID,),
    in_specs=[
        pl.BlockSpec((C, _BLK), lambda i: (0, i)),
        pl.BlockSpec((D, _BLK), lambda i: (0, i)),
        pl.BlockSpec((C, D), lambda i: (0, 0)),
        pl.BlockSpec((C, 1), lambda i: (0, 0)),
    ],
    out_specs=pl.BlockSpec((_BLK, C), lambda i: (i, 0)),
    out_shape=jax.ShapeDtypeStruct((NP, C), jnp.float32),
)


@jax.jit
def kernel(in_feat, edge_index, W_pool1, b_pool1, W_self1, W_neigh1, b_neigh1,
           W_pool2, b_pool2, W_self2, W_neigh2, b_neigh2):
    src = edge_index[0].astype(jnp.int32)
    dst = edge_index[1].astype(jnp.int32)

    xpad = jnp.pad(in_feat, ((0, NP - N), (0, 0)))
    m1t, s1t = _dense1(xpad, W_pool1, b_pool1.reshape(D, 1), W_self1)
    agg1t = _segmax(m1t.reshape(-1), src, dst).reshape(D, NP)
    m2t, s2t = _dense2(s1t, agg1t, W_neigh1, b_neigh1.reshape(D, 1),
                       W_pool2, b_pool2.reshape(D, 1), W_self2)
    agg2t = _segmax(m2t.reshape(-1), src, dst).reshape(D, NP)
    return _dense3(s2t, agg2t, W_neigh2, b_neigh2.reshape(C, 1))[:N]
